# Initial kernel scaffold; baseline (speedup 1.0000x reference)
import jax
import jax.numpy as jnp
from jax.experimental import pallas as pl


def kernel(x_gene, x_cell, edge_index_gg, edge_index_cc, edge_index_cg, edge_index_gc, Wl_gg, bl_gg, Wr_gg, Wl_cc, bl_cc, Wr_cc, Wl_cg, bl_cg, Wr_cg, Wl_gc, bl_gc, Wr_gc, W_gene, b_gene, W_cell, b_cell):
    raise NotImplementedError("write your pallas kernel here")


# speedup vs baseline: 8.6346x; 8.6346x over previous
import functools

import jax
import jax.numpy as jnp
from jax import lax
from jax.experimental import pallas as pl
from jax.experimental.pallas import tpu as pltpu
from jax.experimental.pallas import tpu_sc as plsc

N = 100000
NP = 100096
SINK = 100000
E = 1600000
NC = 2
NS = 16
NW = NC * NS
RW = 392
R = NW * RW
EP = R * 128
TPB = NP // NS
H = 64
OUT = 2


@functools.cache
def _sc_aggregate():
    mesh = plsc.VectorSubcoreMesh(core_axis_name="c", subcore_axis_name="s",
                                  num_cores=NC, num_subcores=NS)
    out_ty = [jax.ShapeDtypeStruct((NC, NP, 8), jnp.float32) for _ in range(2)]
    scratch = [
        pltpu.VMEM((128,), jnp.int32),
        pltpu.VMEM((128,), jnp.int32),
        pltpu.VMEM((128, 8), jnp.float32),
        pltpu.SemaphoreType.DMA,
        pltpu.VMEM_SHARED((NP, 8), jnp.float32),
        pltpu.VMEM_SHARED((NP, 8), jnp.float32),
    ]

    @functools.partial(
        pl.kernel, out_type=out_ty, mesh=mesh, scratch_types=scratch,
        compiler_params=pltpu.CompilerParams(use_tc_tiling_on_sc=False))
    def k(xg_lo, xc_hi, xc_lo, xg_hi, zer,
          sgg, dgg, scg, dcg, scc, dcc, sgc, dgc,
          og, oc,
          sidx, didx, rows, sem, accg, accc):
        c = lax.axis_index("c")
        s = lax.axis_index("s")
        wid = s * NC + c
        rels = (
            (xg_lo, sgg, dgg, accg),
            (xc_hi, scg, dcg, accg),
            (xc_lo, scc, dcc, accc),
            (xg_hi, sgc, dgc, accc),
        )

        zoff = s * TPB
        pltpu.sync_copy(zer, accg.at[pl.ds(zoff, TPB)])
        pltpu.sync_copy(zer, accc.at[pl.ds(zoff, TPB)])
        plsc.subcore_barrier()

        base = wid * RW
        for tab, srcr, dstr, acc in rels:
            def rowstep(i, _, tab=tab, srcr=srcr, dstr=dstr, acc=acc):
                off = (base + i) * 128
                pltpu.sync_copy(srcr.at[pl.ds(off, 128)], sidx)
                pltpu.sync_copy(dstr.at[pl.ds(off, 128)], didx)
                pltpu.async_copy(tab.at[sidx], rows, sem).wait()
                pltpu.sync_copy(rows, acc.at[didx], add=True)
                return 0

            lax.fori_loop(0, RW, rowstep, 0)

        plsc.subcore_barrier()
        pltpu.sync_copy(accg.at[pl.ds(zoff, TPB)], og.at[c, pl.ds(zoff, TPB)])
        pltpu.sync_copy(accc.at[pl.ds(zoff, TPB)], oc.at[c, pl.ds(zoff, TPB)])

    return k


def _dense_body(p_ref, x_ref, a_ref, w_ref, b_ref, o_ref):
    sum8 = p_ref[0] + p_ref[1]
    sa = sum8[:, 0:4]
    sb = sum8[:, 4:8]
    ma = sa / jnp.maximum(sa[:, 3:4], 1.0)
    mb = sb / jnp.maximum(sb[:, 3:4], 1.0)
    a = a_ref[...]
    dot = functools.partial(jax.lax.dot_general,
                            dimension_numbers=(((1,), (0,)), ((), ())),
                            preferred_element_type=jnp.float32)
    pre = dot(ma, a[0:4]) + dot(mb, a[4:8]) + dot(x_ref[:, 0:4], a[8:12])
    h = jnp.maximum(pre, 0.0)
    o_ref[...] = dot(h, w_ref[...]) + b_ref[...]


_DB = TPB


def _dense(p, x, a, w, b):
    grid = NP // _DB
    return pl.pallas_call(
        _dense_body,
        grid=(grid,),
        in_specs=[
            pl.BlockSpec((NC, _DB, 8), lambda i: (0, i, 0)),
            pl.BlockSpec((_DB, 8), lambda i: (i, 0)),
            pl.BlockSpec((12, H), lambda i: (0, 0)),
            pl.BlockSpec((H, OUT), lambda i: (0, 0)),
            pl.BlockSpec((1, OUT), lambda i: (0, 0)),
        ],
        out_specs=pl.BlockSpec((_DB, OUT), lambda i: (i, 0)),
        out_shape=jax.ShapeDtypeStruct((NP, OUT), jnp.float32),
    )(p, x, a, w, b)


def _pad_nodes(x, hi):
    lo4 = jnp.concatenate([x, jnp.ones((N, 1), jnp.float32)], axis=1)
    z4 = jnp.zeros((N, 4), jnp.float32)
    row = jnp.concatenate([z4, lo4] if hi else [lo4, z4], axis=1)
    return jnp.concatenate([row, jnp.zeros((NP - N, 8), jnp.float32)], axis=0)


def _prep_edges(ei):
    pad_src = jnp.zeros((EP - E,), jnp.int32)
    pad_dst = jnp.full((EP - E,), SINK, jnp.int32)
    src = jnp.concatenate([ei[0], pad_src])
    dst = jnp.concatenate([ei[1], pad_dst])
    return src, dst


def _combine_w(wl_a, bl_a, wl_b, bl_b, wr_a, wr_b):
    z = jnp.zeros((1, H), jnp.float32)
    return 0.5 * jnp.concatenate([
        wl_a, z, wl_b, z, wr_a + wr_b, (bl_a + bl_b)[None, :]], axis=0)


def kernel(x_gene, x_cell, edge_index_gg, edge_index_cc, edge_index_cg,
           edge_index_gc, Wl_gg, bl_gg, Wr_gg, Wl_cc, bl_cc, Wr_cc,
           Wl_cg, bl_cg, Wr_cg, Wl_gc, bl_gc, Wr_gc,
           W_gene, b_gene, W_cell, b_cell):
    xg_lo = _pad_nodes(x_gene, hi=False)
    xg_hi = _pad_nodes(x_gene, hi=True)
    xc_lo = _pad_nodes(x_cell, hi=False)
    xc_hi = _pad_nodes(x_cell, hi=True)
    zer = jnp.zeros((TPB, 8), jnp.float32)
    sgg, dgg = _prep_edges(edge_index_gg)
    scg, dcg = _prep_edges(edge_index_cg)
    scc, dcc = _prep_edges(edge_index_cc)
    sgc, dgc = _prep_edges(edge_index_gc)

    pg, pc = _sc_aggregate()(
        xg_lo, xc_hi, xc_lo, xg_hi, zer,
        sgg, dgg, scg, dcg, scc, dcc, sgc, dgc)

    a_gene = _combine_w(Wl_gg, bl_gg, Wl_cg, bl_cg, Wr_gg, Wr_cg)
    a_cell = _combine_w(Wl_cc, bl_cc, Wl_gc, bl_gc, Wr_cc, Wr_gc)

    out_gene = _dense(pg, xg_lo, a_gene, W_gene, b_gene[None, :])
    out_cell = _dense(pc, xc_lo, a_cell, W_cell, b_cell[None, :])
    return (out_gene[:N], out_cell[:N])


# speedup vs baseline: 14.5275x; 1.6825x over previous
import functools

import jax
import jax.numpy as jnp
from jax import lax
from jax.experimental import pallas as pl
from jax.experimental.pallas import tpu as pltpu
from jax.experimental.pallas import tpu_sc as plsc

N = 100000
NP = 100096
SINK = 100000
E = 1600000
NC = 2
NS = 16
NW = NC * NS
KC = 2048
ITERS = 25
EW = KC * ITERS
EP = NW * EW
TPB = NP // NS
H = 64
OUT = 2


@functools.cache
def _sc_aggregate():
    mesh = plsc.VectorSubcoreMesh(core_axis_name="c", subcore_axis_name="s",
                                  num_cores=NC, num_subcores=NS)
    out_ty = [jax.ShapeDtypeStruct((NC, NP, 8), jnp.float32) for _ in range(2)]
    scratch = [
        pltpu.VMEM((KC,), jnp.int32),
        pltpu.VMEM((KC,), jnp.int32),
        pltpu.VMEM((KC, 8), jnp.float32),
        pltpu.SemaphoreType.DMA,
        pltpu.VMEM_SHARED((NP, 8), jnp.float32),
        pltpu.VMEM_SHARED((NP, 8), jnp.float32),
    ]

    @functools.partial(
        pl.kernel, out_type=out_ty, mesh=mesh, scratch_types=scratch,
        compiler_params=pltpu.CompilerParams(use_tc_tiling_on_sc=False))
    def k(xg_lo, xc_hi, xc_lo, xg_hi, zer,
          sgg, dgg, scg, dcg, scc, dcc, sgc, dgc,
          og, oc,
          sidx, didx, rows, sem, accg, accc):
        c = lax.axis_index("c")
        s = lax.axis_index("s")
        wid = s * NC + c
        rels = (
            (xg_lo, sgg, dgg, accg),
            (xc_hi, scg, dcg, accg),
            (xc_lo, scc, dcc, accc),
            (xg_hi, sgc, dgc, accc),
        )

        zoff = s * TPB
        pltpu.sync_copy(zer, accg.at[pl.ds(zoff, TPB)])
        pltpu.sync_copy(zer, accc.at[pl.ds(zoff, TPB)])
        plsc.subcore_barrier()

        base = wid * EW
        for tab, srcr, dstr, acc in rels:
            def rowstep(i, _, tab=tab, srcr=srcr, dstr=dstr, acc=acc):
                off = base + i * KC
                pltpu.sync_copy(srcr.at[pl.ds(off, KC)], sidx)
                pltpu.sync_copy(dstr.at[pl.ds(off, KC)], didx)
                pltpu.async_copy(tab.at[sidx], rows, sem).wait()
                pltpu.sync_copy(rows, acc.at[didx], add=True)
                return 0

            lax.fori_loop(0, ITERS, rowstep, 0)

        plsc.subcore_barrier()
        pltpu.sync_copy(accg.at[pl.ds(zoff, TPB)], og.at[c, pl.ds(zoff, TPB)])
        pltpu.sync_copy(accc.at[pl.ds(zoff, TPB)], oc.at[c, pl.ds(zoff, TPB)])

    return k


def _dense_body(p_ref, x_ref, a_ref, w_ref, b_ref, o_ref):
    sum8 = p_ref[0] + p_ref[1]
    sa = sum8[:, 0:4]
    sb = sum8[:, 4:8]
    ma = sa / jnp.maximum(sa[:, 3:4], 1.0)
    mb = sb / jnp.maximum(sb[:, 3:4], 1.0)
    a = a_ref[...]
    dot = functools.partial(jax.lax.dot_general,
                            dimension_numbers=(((1,), (0,)), ((), ())),
                            preferred_element_type=jnp.float32)
    pre = dot(ma, a[0:4]) + dot(mb, a[4:8]) + dot(x_ref[:, 0:4], a[8:12])
    h = jnp.maximum(pre, 0.0)
    o_ref[...] = dot(h, w_ref[...]) + b_ref[...]


_DB = TPB


def _dense(p, x, a, w, b):
    grid = NP // _DB
    return pl.pallas_call(
        _dense_body,
        grid=(grid,),
        in_specs=[
            pl.BlockSpec((NC, _DB, 8), lambda i: (0, i, 0)),
            pl.BlockSpec((_DB, 8), lambda i: (i, 0)),
            pl.BlockSpec((12, H), lambda i: (0, 0)),
            pl.BlockSpec((H, OUT), lambda i: (0, 0)),
            pl.BlockSpec((1, OUT), lambda i: (0, 0)),
        ],
        out_specs=pl.BlockSpec((_DB, OUT), lambda i: (i, 0)),
        out_shape=jax.ShapeDtypeStruct((NP, OUT), jnp.float32),
    )(p, x, a, w, b)


def _pad_nodes(x, hi):
    lo4 = jnp.concatenate([x, jnp.ones((N, 1), jnp.float32)], axis=1)
    z4 = jnp.zeros((N, 4), jnp.float32)
    row = jnp.concatenate([z4, lo4] if hi else [lo4, z4], axis=1)
    return jnp.concatenate([row, jnp.zeros((NP - N, 8), jnp.float32)], axis=0)


def _prep_edges(ei):
    pad_src = jnp.zeros((EP - E,), jnp.int32)
    pad_dst = jnp.full((EP - E,), SINK, jnp.int32)
    src = jnp.concatenate([ei[0], pad_src])
    dst = jnp.concatenate([ei[1], pad_dst])
    return src, dst


def _combine_w(wl_a, bl_a, wl_b, bl_b, wr_a, wr_b):
    z = jnp.zeros((1, H), jnp.float32)
    return 0.5 * jnp.concatenate([
        wl_a, z, wl_b, z, wr_a + wr_b, (bl_a + bl_b)[None, :]], axis=0)


def kernel(x_gene, x_cell, edge_index_gg, edge_index_cc, edge_index_cg,
           edge_index_gc, Wl_gg, bl_gg, Wr_gg, Wl_cc, bl_cc, Wr_cc,
           Wl_cg, bl_cg, Wr_cg, Wl_gc, bl_gc, Wr_gc,
           W_gene, b_gene, W_cell, b_cell):
    xg_lo = _pad_nodes(x_gene, hi=False)
    xg_hi = _pad_nodes(x_gene, hi=True)
    xc_lo = _pad_nodes(x_cell, hi=False)
    xc_hi = _pad_nodes(x_cell, hi=True)
    zer = jnp.zeros((TPB, 8), jnp.float32)
    sgg, dgg = _prep_edges(edge_index_gg)
    scg, dcg = _prep_edges(edge_index_cg)
    scc, dcc = _prep_edges(edge_index_cc)
    sgc, dgc = _prep_edges(edge_index_gc)

    pg, pc = _sc_aggregate()(
        xg_lo, xc_hi, xc_lo, xg_hi, zer,
        sgg, dgg, scg, dcg, scc, dcc, sgc, dgc)

    a_gene = _combine_w(Wl_gg, bl_gg, Wl_cg, bl_cg, Wr_gg, Wr_cg)
    a_cell = _combine_w(Wl_cc, bl_cc, Wl_gc, bl_gc, Wr_cc, Wr_gc)

    out_gene = _dense(pg, xg_lo, a_gene, W_gene, b_gene[None, :])
    out_cell = _dense(pc, xc_lo, a_cell, W_cell, b_cell[None, :])
    return (out_gene[:N], out_cell[:N])


# speedup vs baseline: 15.8207x; 1.0890x over previous
import functools

import jax
import jax.numpy as jnp
from jax import lax
from jax.experimental import pallas as pl
from jax.experimental.pallas import tpu as pltpu
from jax.experimental.pallas import tpu_sc as plsc

N = 100000
NP = 100096
SINK = 100000
E = 1600000
NC = 2
NS = 16
NW = NC * NS
KC = 2048
ITERS = 25
EW = KC * ITERS
EP = NW * EW
TPB = NP // NS
H = 64
OUT = 2


@functools.cache
def _sc_aggregate():
    mesh = plsc.VectorSubcoreMesh(core_axis_name="c", subcore_axis_name="s",
                                  num_cores=NC, num_subcores=NS)
    out_ty = [jax.ShapeDtypeStruct((NC, NP, 8), jnp.float32) for _ in range(2)]
    scratch = [
        pltpu.VMEM((KC,), jnp.int32),
        pltpu.VMEM((KC,), jnp.int32),
        pltpu.VMEM((KC, 8), jnp.float32),
        pltpu.SemaphoreType.DMA,
        pltpu.VMEM_SHARED((NP, 8), jnp.float32),
        pltpu.VMEM_SHARED((NP, 8), jnp.float32),
    ]

    @functools.partial(
        pl.kernel, out_type=out_ty, mesh=mesh, scratch_types=scratch,
        compiler_params=pltpu.CompilerParams(use_tc_tiling_on_sc=False))
    def k(xg_lo, xc_hi, xc_lo, xg_hi, zer,
          sgg, dgg, scg, dcg, scc, dcc, sgc, dgc,
          og, oc,
          sidx, didx, rows, sem, accg, accc):
        c = lax.axis_index("c")
        s = lax.axis_index("s")
        wid = s * NC + c
        rels = (
            (xg_lo, sgg, dgg, accg),
            (xc_hi, scg, dcg, accg),
            (xc_lo, scc, dcc, accc),
            (xg_hi, sgc, dgc, accc),
        )

        zoff = s * TPB
        pltpu.sync_copy(zer, accg.at[pl.ds(zoff, TPB)])
        pltpu.sync_copy(zer, accc.at[pl.ds(zoff, TPB)])
        plsc.subcore_barrier()

        base = wid * EW
        for tab, srcr, dstr, acc in rels:
            def rowstep(i, _, tab=tab, srcr=srcr, dstr=dstr, acc=acc):
                off = base + i * KC
                pltpu.sync_copy(srcr.at[pl.ds(off, KC)], sidx)
                pltpu.sync_copy(dstr.at[pl.ds(off, KC)], didx)
                pltpu.async_copy(tab.at[sidx], rows, sem).wait()
                pltpu.sync_copy(rows, acc.at[didx], add=True)
                return 0

            lax.fori_loop(0, ITERS, rowstep, 0)

        plsc.subcore_barrier()
        pltpu.sync_copy(accg.at[pl.ds(zoff, TPB)], og.at[c, pl.ds(zoff, TPB)])
        pltpu.sync_copy(accc.at[pl.ds(zoff, TPB)], oc.at[c, pl.ds(zoff, TPB)])

    return k


def _dense_body(p_ref, x_ref, a_ref, w_ref, b_ref, o_ref):
    sum8 = p_ref[0] + p_ref[1]
    sa = sum8[:, 0:4]
    sb = sum8[:, 4:8]
    ma = sa / jnp.maximum(sa[:, 3:4], 1.0)
    mb = sb / jnp.maximum(sb[:, 3:4], 1.0)
    a = a_ref[...]
    dot = functools.partial(jax.lax.dot_general,
                            dimension_numbers=(((1,), (0,)), ((), ())),
                            preferred_element_type=jnp.float32)
    pre = dot(ma, a[0:4]) + dot(mb, a[4:8]) + dot(x_ref[:, 0:4], a[8:12])
    h = jnp.maximum(pre, 0.0)
    o_ref[...] = dot(h, w_ref[...]) + b_ref[...]


_DB = TPB


def _dense(p, x, a, w, b):
    grid = NP // _DB
    return pl.pallas_call(
        _dense_body,
        grid=(grid,),
        in_specs=[
            pl.BlockSpec((NC, _DB, 8), lambda i: (0, i, 0)),
            pl.BlockSpec((_DB, 8), lambda i: (i, 0)),
            pl.BlockSpec((12, H), lambda i: (0, 0)),
            pl.BlockSpec((H, OUT), lambda i: (0, 0)),
            pl.BlockSpec((1, OUT), lambda i: (0, 0)),
        ],
        out_specs=pl.BlockSpec((_DB, OUT), lambda i: (i, 0)),
        out_shape=jax.ShapeDtypeStruct((NP, OUT), jnp.float32),
    )(p, x, a, w, b)


def _pad_nodes(x, hi):
    lo4 = jnp.concatenate([x, jnp.ones((N, 1), jnp.float32)], axis=1)
    z4 = jnp.zeros((N, 4), jnp.float32)
    row = jnp.concatenate([z4, lo4] if hi else [lo4, z4], axis=1)
    return jnp.concatenate([row, jnp.zeros((NP - N, 8), jnp.float32)], axis=0)


def _prep_edges(ei):
    pad_src = jnp.zeros((EP - E,), jnp.int32)
    pad_dst = SINK + jnp.arange(EP - E, dtype=jnp.int32) % (NP - N)
    src = jnp.concatenate([ei[0], pad_src])
    dst = jnp.concatenate([ei[1], pad_dst])
    return src, dst


def _combine_w(wl_a, bl_a, wl_b, bl_b, wr_a, wr_b):
    z = jnp.zeros((1, H), jnp.float32)
    return 0.5 * jnp.concatenate([
        wl_a, z, wl_b, z, wr_a + wr_b, (bl_a + bl_b)[None, :]], axis=0)


def kernel(x_gene, x_cell, edge_index_gg, edge_index_cc, edge_index_cg,
           edge_index_gc, Wl_gg, bl_gg, Wr_gg, Wl_cc, bl_cc, Wr_cc,
           Wl_cg, bl_cg, Wr_cg, Wl_gc, bl_gc, Wr_gc,
           W_gene, b_gene, W_cell, b_cell):
    xg_lo = _pad_nodes(x_gene, hi=False)
    xg_hi = _pad_nodes(x_gene, hi=True)
    xc_lo = _pad_nodes(x_cell, hi=False)
    xc_hi = _pad_nodes(x_cell, hi=True)
    zer = jnp.zeros((TPB, 8), jnp.float32)
    sgg, dgg = _prep_edges(edge_index_gg)
    scg, dcg = _prep_edges(edge_index_cg)
    scc, dcc = _prep_edges(edge_index_cc)
    sgc, dgc = _prep_edges(edge_index_gc)

    pg, pc = _sc_aggregate()(
        xg_lo, xc_hi, xc_lo, xg_hi, zer,
        sgg, dgg, scg, dcg, scc, dcc, sgc, dgc)

    a_gene = _combine_w(Wl_gg, bl_gg, Wl_cg, bl_cg, Wr_gg, Wr_cg)
    a_cell = _combine_w(Wl_cc, bl_cc, Wl_gc, bl_gc, Wr_cc, Wr_gc)

    out_gene = _dense(pg, xg_lo, a_gene, W_gene, b_gene[None, :])
    out_cell = _dense(pc, xc_lo, a_cell, W_cell, b_cell[None, :])
    return (out_gene[:N], out_cell[:N])


# speedup vs baseline: 21.3709x; 1.3508x over previous
import functools

import jax
import jax.numpy as jnp
from jax import lax
from jax.experimental import pallas as pl
from jax.experimental.pallas import tpu as pltpu
from jax.experimental.pallas import tpu_sc as plsc

N = 100000
NP = 100096
SINK = 100000
E = 1600000
NC = 2
NS = 16
NW = NC * NS
KC = 2000
ITERS = 25
EW = KC * ITERS
TPB = NP // NS
H = 64
OUT = 2


@functools.cache
def _sc_aggregate():
    mesh = plsc.VectorSubcoreMesh(core_axis_name="c", subcore_axis_name="s",
                                  num_cores=NC, num_subcores=NS)
    out_ty = [jax.ShapeDtypeStruct((NC, NP, 8), jnp.float32) for _ in range(2)]
    scratch = [
        pltpu.VMEM((KC,), jnp.int32),
        pltpu.VMEM((KC,), jnp.int32),
        pltpu.VMEM((KC, 8), jnp.float32),
        pltpu.SemaphoreType.DMA,
        pltpu.VMEM_SHARED((NP, 8), jnp.float32),
        pltpu.VMEM_SHARED((NP, 8), jnp.float32),
    ]

    @functools.partial(
        pl.kernel, out_type=out_ty, mesh=mesh, scratch_types=scratch,
        compiler_params=pltpu.CompilerParams(use_tc_tiling_on_sc=False))
    def k(xg_lo, xc_hi, xc_lo, xg_hi, zer,
          sgg, dgg, scg, dcg, scc, dcc, sgc, dgc,
          og, oc,
          sidx, didx, rows, sem, accg, accc):
        c = lax.axis_index("c")
        s = lax.axis_index("s")
        wid = s * NC + c
        rels = (
            (xg_lo, sgg, dgg, accg),
            (xc_hi, scg, dcg, accg),
            (xc_lo, scc, dcc, accc),
            (xg_hi, sgc, dgc, accc),
        )

        zoff = s * TPB
        pltpu.sync_copy(zer, accg.at[pl.ds(zoff, TPB)])
        pltpu.sync_copy(zer, accc.at[pl.ds(zoff, TPB)])
        plsc.subcore_barrier()

        base = wid * EW
        for tab, srcr, dstr, acc in rels:
            def rowstep(i, _, tab=tab, srcr=srcr, dstr=dstr, acc=acc):
                off = base + i * KC
                pltpu.sync_copy(srcr.at[pl.ds(off, KC)], sidx)
                pltpu.sync_copy(dstr.at[pl.ds(off, KC)], didx)
                pltpu.async_copy(tab.at[sidx], rows, sem).wait()
                pltpu.sync_copy(rows, acc.at[didx], add=True)
                return 0

            lax.fori_loop(0, ITERS, rowstep, 0)

        plsc.subcore_barrier()
        pltpu.sync_copy(accg.at[pl.ds(zoff, TPB)], og.at[c, pl.ds(zoff, TPB)])
        pltpu.sync_copy(accc.at[pl.ds(zoff, TPB)], oc.at[c, pl.ds(zoff, TPB)])

    return k


def _dense_body(p_ref, x_ref, a_ref, w_ref, b_ref, o_ref):
    sum8 = p_ref[0] + p_ref[1]
    sa = sum8[:, 0:4]
    sb = sum8[:, 4:8]
    ma = sa / jnp.maximum(sa[:, 3:4], 1.0)
    mb = sb / jnp.maximum(sb[:, 3:4], 1.0)
    a = a_ref[...]
    dot = functools.partial(jax.lax.dot_general,
                            dimension_numbers=(((1,), (0,)), ((), ())),
                            preferred_element_type=jnp.float32)
    pre = dot(ma, a[0:4]) + dot(mb, a[4:8]) + dot(x_ref[:, 0:4], a[8:12])
    h = jnp.maximum(pre, 0.0)
    o_ref[...] = dot(h, w_ref[...]) + b_ref[...]


_DB = TPB


def _dense(p, x, a, w, b):
    grid = NP // _DB
    return pl.pallas_call(
        _dense_body,
        grid=(grid,),
        in_specs=[
            pl.BlockSpec((NC, _DB, 8), lambda i: (0, i, 0)),
            pl.BlockSpec((_DB, 8), lambda i: (i, 0)),
            pl.BlockSpec((12, H), lambda i: (0, 0)),
            pl.BlockSpec((H, OUT), lambda i: (0, 0)),
            pl.BlockSpec((1, OUT), lambda i: (0, 0)),
        ],
        out_specs=pl.BlockSpec((_DB, OUT), lambda i: (i, 0)),
        out_shape=jax.ShapeDtypeStruct((NP, OUT), jnp.float32),
    )(p, x, a, w, b)


def _pad_nodes(x, hi):
    lo4 = jnp.concatenate([x, jnp.ones((N, 1), jnp.float32)], axis=1)
    z4 = jnp.zeros((N, 4), jnp.float32)
    row = jnp.concatenate([z4, lo4] if hi else [lo4, z4], axis=1)
    return jnp.concatenate([row, jnp.zeros((NP - N, 8), jnp.float32)], axis=0)


def _prep_edges(ei):
    return ei[0], ei[1]


def _combine_w(wl_a, bl_a, wl_b, bl_b, wr_a, wr_b):
    z = jnp.zeros((1, H), jnp.float32)
    return 0.5 * jnp.concatenate([
        wl_a, z, wl_b, z, wr_a + wr_b, (bl_a + bl_b)[None, :]], axis=0)


def kernel(x_gene, x_cell, edge_index_gg, edge_index_cc, edge_index_cg,
           edge_index_gc, Wl_gg, bl_gg, Wr_gg, Wl_cc, bl_cc, Wr_cc,
           Wl_cg, bl_cg, Wr_cg, Wl_gc, bl_gc, Wr_gc,
           W_gene, b_gene, W_cell, b_cell):
    xg_lo = _pad_nodes(x_gene, hi=False)
    xg_hi = _pad_nodes(x_gene, hi=True)
    xc_lo = _pad_nodes(x_cell, hi=False)
    xc_hi = _pad_nodes(x_cell, hi=True)
    zer = jnp.zeros((TPB, 8), jnp.float32)
    sgg, dgg = _prep_edges(edge_index_gg)
    scg, dcg = _prep_edges(edge_index_cg)
    scc, dcc = _prep_edges(edge_index_cc)
    sgc, dgc = _prep_edges(edge_index_gc)

    pg, pc = _sc_aggregate()(
        xg_lo, xc_hi, xc_lo, xg_hi, zer,
        sgg, dgg, scg, dcg, scc, dcc, sgc, dgc)

    a_gene = _combine_w(Wl_gg, bl_gg, Wl_cg, bl_cg, Wr_gg, Wr_cg)
    a_cell = _combine_w(Wl_cc, bl_cc, Wl_gc, bl_gc, Wr_cc, Wr_gc)

    out_gene = _dense(pg, xg_lo, a_gene, W_gene, b_gene[None, :])
    out_cell = _dense(pc, xc_lo, a_cell, W_cell, b_cell[None, :])
    return (out_gene[:N], out_cell[:N])


# speedup vs baseline: 21.7284x; 1.0167x over previous
import functools

import jax
import jax.numpy as jnp
from jax import lax
from jax.experimental import pallas as pl
from jax.experimental.pallas import tpu as pltpu
from jax.experimental.pallas import tpu_sc as plsc

N = 100000
NP = 100096
SINK = 100000
E = 1600000
NC = 2
NS = 16
NW = NC * NS
KC = 2000
ITERS = 25
EW = KC * ITERS
TPB = NP // NS
H = 64
OUT = 2


@functools.cache
def _sc_aggregate():
    mesh = plsc.VectorSubcoreMesh(core_axis_name="c", subcore_axis_name="s",
                                  num_cores=NC, num_subcores=NS)
    out_ty = [jax.ShapeDtypeStruct((NC, NP, 8), jnp.float32) for _ in range(2)]
    scratch = [
        pltpu.VMEM((KC,), jnp.int32),
        pltpu.VMEM((KC,), jnp.int32),
        pltpu.VMEM((KC, 8), jnp.float32),
        pltpu.SemaphoreType.DMA,
        pltpu.VMEM_SHARED((NP, 8), jnp.float32),
        pltpu.VMEM_SHARED((NP, 8), jnp.float32),
    ]

    @functools.partial(
        pl.kernel, out_type=out_ty, mesh=mesh, scratch_types=scratch,
        compiler_params=pltpu.CompilerParams(use_tc_tiling_on_sc=False))
    def k(xg_lo, xc_hi, xc_lo, xg_hi, zer,
          sgg, dgg, scg, dcg, scc, dcc, sgc, dgc,
          og, oc,
          sidx, didx, rows, sem, accg, accc):
        c = lax.axis_index("c")
        s = lax.axis_index("s")
        wid = s * NC + c
        rels = (
            (xg_lo, sgg, dgg, accg),
            (xc_hi, scg, dcg, accg),
            (xc_lo, scc, dcc, accc),
            (xg_hi, sgc, dgc, accc),
        )

        zoff = s * TPB
        pltpu.sync_copy(zer, accg.at[pl.ds(zoff, TPB)])
        pltpu.sync_copy(zer, accc.at[pl.ds(zoff, TPB)])
        plsc.subcore_barrier()

        base = wid * EW
        for tab, srcr, dstr, acc in rels:
            def rowstep(i, _, tab=tab, srcr=srcr, dstr=dstr, acc=acc):
                off = base + i * KC
                pltpu.sync_copy(srcr.at[pl.ds(off, KC)], sidx)
                pltpu.sync_copy(dstr.at[pl.ds(off, KC)], didx)
                pltpu.async_copy(tab.at[sidx], rows, sem).wait()
                pltpu.sync_copy(rows, acc.at[didx], add=True)
                return 0

            lax.fori_loop(0, ITERS, rowstep, 0)

        plsc.subcore_barrier()
        pltpu.sync_copy(accg.at[pl.ds(zoff, TPB)], og.at[c, pl.ds(zoff, TPB)])
        pltpu.sync_copy(accc.at[pl.ds(zoff, TPB)], oc.at[c, pl.ds(zoff, TPB)])

    return k


def _dense_body(p_ref, x_ref, a_ref, w_ref, b_ref, o_ref):
    sum8 = p_ref[0] + p_ref[1]
    sa = sum8[:, 0:4]
    sb = sum8[:, 4:8]
    ma = sa / jnp.maximum(sa[:, 3:4], 1.0)
    mb = sb / jnp.maximum(sb[:, 3:4], 1.0)
    a = a_ref[...]
    dot = functools.partial(jax.lax.dot_general,
                            dimension_numbers=(((1,), (0,)), ((), ())),
                            preferred_element_type=jnp.float32)
    pre = (dot(ma, a[0:4]) + dot(mb, a[4:8]) + dot(x_ref[...], a[8:11])
           + a[11:12])
    h = jnp.maximum(pre, 0.0)
    o_ref[...] = dot(h, w_ref[...]) + b_ref[...]


_DB = 4000


def _dense(p, x, a, w, b):
    grid = N // _DB
    return pl.pallas_call(
        _dense_body,
        grid=(grid,),
        in_specs=[
            pl.BlockSpec((NC, _DB, 8), lambda i: (0, i, 0)),
            pl.BlockSpec((_DB, 3), lambda i: (i, 0)),
            pl.BlockSpec((12, H), lambda i: (0, 0)),
            pl.BlockSpec((H, OUT), lambda i: (0, 0)),
            pl.BlockSpec((1, OUT), lambda i: (0, 0)),
        ],
        out_specs=pl.BlockSpec((_DB, OUT), lambda i: (i, 0)),
        out_shape=jax.ShapeDtypeStruct((N, OUT), jnp.float32),
    )(p, x, a, w, b)


def _pad_nodes(x, hi):
    lo4 = jnp.concatenate([x, jnp.ones((N, 1), jnp.float32)], axis=1)
    z4 = jnp.zeros((N, 4), jnp.float32)
    row = jnp.concatenate([z4, lo4] if hi else [lo4, z4], axis=1)
    return jnp.concatenate([row, jnp.zeros((NP - N, 8), jnp.float32)], axis=0)


def _prep_edges(ei):
    return ei[0], ei[1]


def _combine_w(wl_a, bl_a, wl_b, bl_b, wr_a, wr_b):
    z = jnp.zeros((1, H), jnp.float32)
    return 0.5 * jnp.concatenate([
        wl_a, z, wl_b, z, wr_a + wr_b, (bl_a + bl_b)[None, :]], axis=0)


def kernel(x_gene, x_cell, edge_index_gg, edge_index_cc, edge_index_cg,
           edge_index_gc, Wl_gg, bl_gg, Wr_gg, Wl_cc, bl_cc, Wr_cc,
           Wl_cg, bl_cg, Wr_cg, Wl_gc, bl_gc, Wr_gc,
           W_gene, b_gene, W_cell, b_cell):
    xg_lo = _pad_nodes(x_gene, hi=False)
    xg_hi = _pad_nodes(x_gene, hi=True)
    xc_lo = _pad_nodes(x_cell, hi=False)
    xc_hi = _pad_nodes(x_cell, hi=True)
    zer = jnp.zeros((TPB, 8), jnp.float32)
    sgg, dgg = _prep_edges(edge_index_gg)
    scg, dcg = _prep_edges(edge_index_cg)
    scc, dcc = _prep_edges(edge_index_cc)
    sgc, dgc = _prep_edges(edge_index_gc)

    pg, pc = _sc_aggregate()(
        xg_lo, xc_hi, xc_lo, xg_hi, zer,
        sgg, dgg, scg, dcg, scc, dcc, sgc, dgc)

    a_gene = _combine_w(Wl_gg, bl_gg, Wl_cg, bl_cg, Wr_gg, Wr_cg)
    a_cell = _combine_w(Wl_cc, bl_cc, Wl_gc, bl_gc, Wr_cc, Wr_gc)

    out_gene = _dense(pg, x_gene, a_gene, W_gene, b_gene[None, :])
    out_cell = _dense(pc, x_cell, a_cell, W_cell, b_cell[None, :])
    return (out_gene, out_cell)


# speedup vs baseline: 28.1337x; 1.2948x over previous
import functools

import jax
import jax.numpy as jnp
from jax import lax
from jax.experimental import pallas as pl
from jax.experimental.pallas import tpu as pltpu
from jax.experimental.pallas import tpu_sc as plsc

N = 100000
NP = 100096
SINK = 100000
E = 1600000
NC = 2
NS = 16
NW = NC * NS
KC = 2000
ITERS = 25
EW = KC * ITERS
TPB = NP // NS
H = 64
OUT = 2


@functools.cache
def _sc_aggregate():
    mesh = plsc.VectorSubcoreMesh(core_axis_name="c", subcore_axis_name="s",
                                  num_cores=NC, num_subcores=NS)
    out_ty = [jax.ShapeDtypeStruct((NC, NP, 8), jnp.float32) for _ in range(2)]
    scratch = [
        pltpu.VMEM((KC,), jnp.int32),
        pltpu.VMEM((KC,), jnp.int32),
        pltpu.VMEM((KC, 8), jnp.float32),
        pltpu.SemaphoreType.DMA,
        pltpu.VMEM_SHARED((NP, 8), jnp.float32),
        pltpu.VMEM_SHARED((NP, 8), jnp.float32),
    ]

    @functools.partial(
        pl.kernel, out_type=out_ty, mesh=mesh, scratch_types=scratch,
        compiler_params=pltpu.CompilerParams(use_tc_tiling_on_sc=False))
    def k(xg_lo, xc_hi, xc_lo, xg_hi, zer,
          sgg, dgg, scg, dcg, scc, dcc, sgc, dgc,
          og, oc,
          sidx, didx, rows, sem, accg, accc):
        c = lax.axis_index("c")
        s = lax.axis_index("s")
        wid = s * NC + c
        rels = (
            (xg_lo, sgg, dgg, accg),
            (xc_hi, scg, dcg, accg),
            (xc_lo, scc, dcc, accc),
            (xg_hi, sgc, dgc, accc),
        )

        zoff = s * TPB
        pltpu.sync_copy(zer, accg.at[pl.ds(zoff, TPB)])
        pltpu.sync_copy(zer, accc.at[pl.ds(zoff, TPB)])
        plsc.subcore_barrier()

        base = wid * EW
        for tab, srcr, dstr, acc in rels:
            def rowstep(i, _, tab=tab, srcr=srcr, dstr=dstr, acc=acc):
                off = base + i * KC
                pltpu.sync_copy(srcr.at[pl.ds(off, KC)], sidx)
                pltpu.sync_copy(dstr.at[pl.ds(off, KC)], didx)
                pltpu.async_copy(tab.at[sidx], rows, sem).wait()
                pltpu.sync_copy(rows, acc.at[didx], add=True)
                return 0

            lax.fori_loop(0, ITERS, rowstep, 0)

        plsc.subcore_barrier()
        pltpu.sync_copy(accg.at[pl.ds(zoff, TPB)], og.at[c, pl.ds(zoff, TPB)])
        pltpu.sync_copy(accc.at[pl.ds(zoff, TPB)], oc.at[c, pl.ds(zoff, TPB)])

    return k


def _dense_body(p_ref, x_ref, a_ref, w_ref, b_ref, o_ref):
    sum8 = p_ref[0] + p_ref[1]
    sa = sum8[:, 0:4]
    sb = sum8[:, 4:8]
    ma = sa / jnp.maximum(sa[:, 3:4], 1.0)
    mb = sb / jnp.maximum(sb[:, 3:4], 1.0)
    a = a_ref[...]
    dot = functools.partial(jax.lax.dot_general,
                            dimension_numbers=(((1,), (0,)), ((), ())),
                            preferred_element_type=jnp.float32)
    pre = (dot(ma, a[0:4]) + dot(mb, a[4:8]) + dot(x_ref[...], a[8:11])
           + a[11:12])
    h = jnp.maximum(pre, 0.0)
    o_ref[...] = dot(h, w_ref[...]) + b_ref[...]


_DB = 4000


def _dense(p, x, a, w, b):
    grid = N // _DB
    return pl.pallas_call(
        _dense_body,
        grid=(grid,),
        in_specs=[
            pl.BlockSpec((NC, _DB, 8), lambda i: (0, i, 0)),
            pl.BlockSpec((_DB, 3), lambda i: (i, 0)),
            pl.BlockSpec((12, H), lambda i: (0, 0)),
            pl.BlockSpec((H, OUT), lambda i: (0, 0)),
            pl.BlockSpec((1, OUT), lambda i: (0, 0)),
        ],
        out_specs=pl.BlockSpec((_DB, OUT), lambda i: (i, 0)),
        out_shape=jax.ShapeDtypeStruct((N, OUT), jnp.float32),
    )(p, x, a, w, b)


def _edge_body(e_ref, s_ref, d_ref):
    s_ref[...] = e_ref[0]
    d_ref[...] = e_ref[1]


def _split_edges(ei):
    return pl.pallas_call(
        _edge_body,
        out_shape=[jax.ShapeDtypeStruct((E,), jnp.int32)] * 2,
    )(ei)


_BT = 4000


def _table_body(xg_ref, xc_ref, *out_refs):
    ones = jnp.ones((_BT, 1), jnp.float32)
    z4 = jnp.zeros((_BT, 4), jnp.float32)
    for x_ref, (lo_ref, hi_ref) in ((xg_ref, out_refs[0:2]),
                                    (xc_ref, out_refs[2:4])):
        x3 = x_ref[...]
        lo_ref[...] = jnp.concatenate([x3, ones, z4], axis=1)
        hi_ref[...] = jnp.concatenate([z4, x3, ones], axis=1)


def _build_tables(xg, xc):
    grid = N // _BT
    return pl.pallas_call(
        _table_body,
        grid=(grid,),
        in_specs=[pl.BlockSpec((_BT, 3), lambda i: (i, 0))] * 2,
        out_specs=[pl.BlockSpec((_BT, 8), lambda i: (i, 0))] * 4,
        out_shape=[jax.ShapeDtypeStruct((N, 8), jnp.float32)] * 4,
    )(xg, xc)


def _combine_w(wl_a, bl_a, wl_b, bl_b, wr_a, wr_b):
    z = jnp.zeros((1, H), jnp.float32)
    return 0.5 * jnp.concatenate([
        wl_a, z, wl_b, z, wr_a + wr_b, (bl_a + bl_b)[None, :]], axis=0)


def kernel(x_gene, x_cell, edge_index_gg, edge_index_cc, edge_index_cg,
           edge_index_gc, Wl_gg, bl_gg, Wr_gg, Wl_cc, bl_cc, Wr_cc,
           Wl_cg, bl_cg, Wr_cg, Wl_gc, bl_gc, Wr_gc,
           W_gene, b_gene, W_cell, b_cell):
    xg_lo, xg_hi, xc_lo, xc_hi = _build_tables(x_gene, x_cell)
    zer = jnp.zeros((TPB, 8), jnp.float32)
    sgg, dgg = _split_edges(edge_index_gg)
    scg, dcg = _split_edges(edge_index_cg)
    scc, dcc = _split_edges(edge_index_cc)
    sgc, dgc = _split_edges(edge_index_gc)

    pg, pc = _sc_aggregate()(
        xg_lo, xc_hi, xc_lo, xg_hi, zer,
        sgg, dgg, scg, dcg, scc, dcc, sgc, dgc)

    a_gene = _combine_w(Wl_gg, bl_gg, Wl_cg, bl_cg, Wr_gg, Wr_cg)
    a_cell = _combine_w(Wl_cc, bl_cc, Wl_gc, bl_gc, Wr_cc, Wr_gc)

    out_gene = _dense(pg, x_gene, a_gene, W_gene, b_gene[None, :])
    out_cell = _dense(pc, x_cell, a_cell, W_cell, b_cell[None, :])
    return (out_gene, out_cell)
